# baseline (speedup 1.0000x reference)
import functools

import jax
import jax.numpy as jnp
from jax import lax
from jax.experimental import pallas as pl
from jax.experimental.pallas import tpu as pltpu
from jax.experimental.pallas import tpu_sc as plsc

BATCH = 128
SEQ = 512
D = 64
VOCAB = 1000000
NC = 2
NS = 16
NW = NC * NS
RPW = BATCH * SEQ // NW
CHUNK = 128
NCHUNK = RPW // CHUNK

CR = 336
CP = CR // 2
NFULL = VOCAB // CR
KMAX = -(-NFULL // NW)
TAIL = VOCAB - NFULL * CR
L = 16


def _sc_compact(table):
    mesh = plsc.VectorSubcoreMesh(core_axis_name="c", subcore_axis_name="s")

    @functools.partial(
        pl.kernel,
        out_type=jax.ShapeDtypeStruct((VOCAB // 2, 2 * D), jnp.float32),
        mesh=mesh,
        scratch_types=[
            pltpu.VMEM((CR, D), jnp.float32),
            pltpu.VMEM((CR, D), jnp.float32),
            pltpu.VMEM((CP, 2 * D), jnp.float32),
            pltpu.VMEM((CP, 2 * D), jnp.float32),
            pltpu.SemaphoreType.DMA,
            pltpu.SemaphoreType.DMA,
            pltpu.SemaphoreType.DMA,
            pltpu.SemaphoreType.DMA,
        ],
        compiler_params=pltpu.CompilerParams(needs_layout_passes=False),
    )
    def k(table_hbm, wide_hbm, a0, a1, b0, b1, ra0, ra1, wa0, wa1):
        wid = lax.axis_index("s") * NC + lax.axis_index("c")
        abufs, rsems = (a0, a1), (ra0, ra1)
        bbufs, wsems = (b0, b1), (wa0, wa1)

        def chunk_of(kk):
            return kk * NW + wid

        def rd(kk):
            pltpu.async_copy(table_hbm.at[pl.ds(chunk_of(kk) * CR, CR)],
                             abufs[kk % 2], rsems[kk % 2])

        def rd_wait(kk):
            pltpu.make_async_copy(table_hbm.at[pl.ds(0, CR)],
                                  abufs[kk % 2], rsems[kk % 2]).wait()

        def wr(kk):
            pltpu.async_copy(bbufs[kk % 2],
                             wide_hbm.at[pl.ds(chunk_of(kk) * CP, CP)],
                             wsems[kk % 2])

        def wr_wait(kk):
            pltpu.make_async_copy(bbufs[kk % 2],
                                  wide_hbm.at[pl.ds(0, CP)],
                                  wsems[kk % 2]).wait()

        def repack(a, b, npairs):
            @pl.loop(0, npairs)
            def _(p):
                for c in range(D // L):
                    b[p, pl.ds(c * L, L)] = a[2 * p, pl.ds(c * L, L)]
                    b[p, pl.ds(D + c * L, L)] = a[2 * p + 1, pl.ds(c * L, L)]

        def valid(kk):
            return chunk_of(kk) < NFULL

        rd(0)
        for kk in range(KMAX):
            full_round = (kk * NW + NW - 1) < NFULL

            def step(kk=kk):
                rd_wait(kk)
                if kk >= 2:
                    wr_wait(kk - 2)
                repack(abufs[kk % 2], bbufs[kk % 2], CP)
                wr(kk)

            if kk + 1 < KMAX:
                if ((kk + 1) * NW + NW - 1) < NFULL:
                    rd(kk + 1)
                else:
                    @pl.when(valid(kk + 1))
                    def _(kk=kk):
                        rd(kk + 1)
            if full_round:
                step()
            else:
                pl.when(valid(kk))(step)
        wr_wait(KMAX - 2)
        wr_wait(KMAX - 1)

        @pl.when(wid == 0)
        def _():
            pltpu.sync_copy(table_hbm.at[pl.ds(NFULL * CR, TAIL)],
                            a0.at[pl.ds(0, TAIL)])
            repack(a0, b0, TAIL // 2)
            pltpu.sync_copy(b0.at[pl.ds(0, TAIL // 2)],
                            wide_hbm.at[pl.ds(NFULL * CP, TAIL // 2)])

    return k(table)


def _sc_gather(pidx3, wide):
    mesh = plsc.VectorSubcoreMesh(core_axis_name="c", subcore_axis_name="s")

    @functools.partial(
        pl.kernel,
        out_type=jax.ShapeDtypeStruct((BATCH * SEQ, 2 * D), jnp.float32),
        mesh=mesh,
        scratch_types=[
            pltpu.VMEM((NCHUNK, CHUNK), jnp.int32),
            pltpu.VMEM((CHUNK, 2 * D), jnp.float32),
            pltpu.VMEM((CHUNK, 2 * D), jnp.float32),
            pltpu.SemaphoreType.DMA,
            pltpu.SemaphoreType.DMA,
        ],
    )
    def k(idx_hbm, wide_hbm, out_hbm, idx_v, rows0, rows1, sem0, sem1):
        wid = lax.axis_index("s") * NC + lax.axis_index("c")
        base = wid * RPW
        pltpu.sync_copy(idx_hbm.at[wid], idx_v)
        bufs = (rows0, rows1)
        sems = (sem0, sem1)
        handles = [None, None]
        handles[0] = pltpu.async_copy(wide_hbm.at[idx_v.at[0]], bufs[0], sems[0])
        for j in range(NCHUNK):
            b = j % 2
            nb = (j + 1) % 2
            if j + 1 < NCHUNK:
                handles[nb] = pltpu.async_copy(
                    wide_hbm.at[idx_v.at[j + 1]], bufs[nb], sems[nb])
            handles[b].wait()
            pltpu.sync_copy(bufs[b], out_hbm.at[pl.ds(base + j * CHUNK, CHUNK)])

    return k(pidx3, wide)


def _tc_select_add_relu(g2, xi, w_pos):
    BB = 8

    def body(g_ref, x_ref, p_ref, o_ref):
        par = (x_ref[...] & 1)[:, :, None]
        lo = g_ref[:, :, :D]
        hi = g_ref[:, :, D:]
        sel = jnp.where(par == 1, hi, lo)
        o_ref[...] = jnp.maximum(sel + p_ref[...][None], 0.0)

    return pl.pallas_call(
        body,
        grid=(BATCH // BB,),
        in_specs=[
            pl.BlockSpec((BB, SEQ, 2 * D), lambda i: (i, 0, 0)),
            pl.BlockSpec((BB, SEQ), lambda i: (i, 0)),
            pl.BlockSpec((SEQ, D), lambda i: (0, 0)),
        ],
        out_specs=pl.BlockSpec((BB, SEQ, D), lambda i: (i, 0, 0)),
        out_shape=jax.ShapeDtypeStruct((BATCH, SEQ, D), jnp.float32),
    )(g2, xi, w_pos)


def kernel(X, W_word, W_pos):
    from jax.experimental.layout import Format, Layout, with_layout_constraint
    xi = X.astype(jnp.int32)
    pidx3 = (xi >> 1).reshape(NW, NCHUNK, CHUNK)
    wide = with_layout_constraint(
        W_word.reshape(VOCAB // 2, 2 * D),
        Layout((0, 1), ((8, 128),)))
    g2 = _sc_gather(pidx3, wide).reshape(BATCH, SEQ, 2 * D)
    return _tc_select_add_relu(g2, xi, W_pos)


# speedup vs baseline: 1.6318x; 1.6318x over previous
import functools

import jax
import jax.numpy as jnp
from jax import lax
from jax.experimental import pallas as pl
from jax.experimental.pallas import tpu as pltpu
from jax.experimental.pallas import tpu_sc as plsc

BATCH = 128
SEQ = 512
D = 64
NC = 2
NS = 16
NW = NC * NS
RPW = BATCH * SEQ // NW
CHUNK = 128
LANES = 16


def _sc_gather(idx2, table):
    mesh = plsc.VectorSubcoreMesh(core_axis_name="c", subcore_axis_name="s")

    @functools.partial(
        pl.kernel,
        out_type=jax.ShapeDtypeStruct((BATCH * SEQ, D), jnp.float32),
        mesh=mesh,
        scratch_types=[
            pltpu.VMEM((RPW,), jnp.int32),
            pltpu.VMEM((CHUNK, D), jnp.float32),
            pltpu.VMEM((CHUNK, D), jnp.float32),
            pltpu.SemaphoreType.DMA,
            pltpu.SemaphoreType.DMA,
            pltpu.SemaphoreType.DMA,
        ],
        compiler_params=pltpu.CompilerParams(needs_layout_passes=False),
    )
    def k(idx_hbm, table_hbm, out_hbm, idx_v, rows0, rows1, isem, gsem, osem):
        wid = lax.axis_index("s") * NC + lax.axis_index("c")
        base = wid * RPW
        pltpu.async_copy(idx_hbm.at[wid], idx_v, isem).wait()
        lane_iota = lax.iota(jnp.int32, LANES)

        def fire(c, buf):
            @pl.loop(0, CHUNK // LANES)
            def _(g):
                v = idx_v[pl.ds(c * CHUNK + g * LANES, LANES)]
                for l in range(LANES):
                    i = jnp.sum(jnp.where(lane_iota == l, v, 0))
                    pltpu.async_copy(table_hbm.at[i],
                                     buf.at[g * LANES + l], gsem)

        def drain_rows(buf):
            pltpu.make_async_copy(table_hbm.at[pl.ds(0, CHUNK)], buf, gsem
                                  ).wait()

        @pl.loop(0, RPW // (2 * CHUNK))
        def _(p):
            c0 = 2 * p
            fire(c0, rows0)
            fire(c0 + 1, rows1)
            drain_rows(rows0)
            pltpu.sync_copy(rows0, out_hbm.at[pl.ds(base + c0 * CHUNK, CHUNK)])
            drain_rows(rows1)
            pltpu.sync_copy(rows1,
                            out_hbm.at[pl.ds(base + (c0 + 1) * CHUNK, CHUNK)])

    return k(idx2, table)


def _tc_add_relu(g, w_pos):
    BB = 8

    def body(g_ref, p_ref, o_ref):
        o_ref[...] = jnp.maximum(g_ref[...] + p_ref[...][None], 0.0)

    return pl.pallas_call(
        body,
        grid=(BATCH // BB,),
        in_specs=[
            pl.BlockSpec((BB, SEQ, D), lambda i: (i, 0, 0)),
            pl.BlockSpec((SEQ, D), lambda i: (0, 0)),
        ],
        out_specs=pl.BlockSpec((BB, SEQ, D), lambda i: (i, 0, 0)),
        out_shape=jax.ShapeDtypeStruct((BATCH, SEQ, D), jnp.float32),
    )(g, w_pos)


def kernel(X, W_word, W_pos):
    idx2 = X.astype(jnp.int32).reshape(NW, RPW)
    g = _sc_gather(idx2, W_word).reshape(BATCH, SEQ, D)
    return _tc_add_relu(g, W_pos)
